# baseline (speedup 1.0000x reference)
import functools

import jax
import jax.numpy as jnp
from jax import lax
from jax.experimental import pallas as pl
from jax.experimental.pallas import tpu as pltpu
from jax.experimental.pallas import tpu_sc as plsc

EMBED_DIM = 32
VOCAB_ROWS = 1000000
BATCH = 16384
N_FIELDS = 26
BBLK = 128
NUM_WORKERS = 32
N_UNITS = N_FIELDS * (BATCH // BBLK)
UPW = N_UNITS // NUM_WORKERS
TBLK = 32768


def _tc_transpose(table_t):
    v = table_t.shape[1]

    def body(in_ref, out_ref):
        eye = (
            lax.broadcasted_iota(jnp.int32, (EMBED_DIM, EMBED_DIM), 0)
            == lax.broadcasted_iota(jnp.int32, (EMBED_DIM, EMBED_DIM), 1)
        ).astype(jnp.float32)
        out_ref[...] = lax.dot_general(
            in_ref[...], eye, (((0,), (0,)), ((), ())),
            preferred_element_type=jnp.float32,
        )

    return pl.pallas_call(
        body,
        grid=(pl.cdiv(v, TBLK),),
        in_specs=[pl.BlockSpec((EMBED_DIM, TBLK), lambda k: (0, k))],
        out_specs=pl.BlockSpec((TBLK, EMBED_DIM), lambda k: (k, 0)),
        out_shape=jax.ShapeDtypeStruct((v, EMBED_DIM), jnp.float32),
    )(table_t)


def _build_gather():
    mesh = plsc.VectorSubcoreMesh(core_axis_name="c", subcore_axis_name="s")
    nbb = BATCH // BBLK

    @functools.partial(
        pl.kernel,
        mesh=mesh,
        compiler_params=pltpu.CompilerParams(
            use_tc_tiling_on_sc=False, needs_layout_passes=False),
        out_type=jax.ShapeDtypeStruct(
            (N_FIELDS, EMBED_DIM // 8, nbb, 8, BBLK), jnp.float32),
        scratch_types=[
            pltpu.VMEM((UPW, BBLK), jnp.int32),
            pltpu.VMEM((BBLK, EMBED_DIM), jnp.float32),
            pltpu.VMEM((EMBED_DIM // 8, 8, BBLK), jnp.float32),
            pltpu.SemaphoreType.DMA,
            pltpu.SemaphoreType.DMA,
        ],
    )
    def gather_kernel(idx_hbm, table_hbm, out_hbm, idx_v, rows_v, w_v,
                      gsem, osem):
        wid = lax.axis_index("s") * 2 + lax.axis_index("c")
        ubase = wid * UPW
        pltpu.sync_copy(idx_hbm.at[pl.ds(ubase, UPW)], idx_v)
        iota16 = lax.iota(jnp.int32, 16)

        def unit_body(g, _):
            c = ubase + g
            f = c // nbb
            bb = lax.rem(c, nbb)
            pltpu.async_copy(table_hbm.at[idx_v.at[g]], rows_v, gsem).wait()
            for b0 in range(0, BBLK, 16):
                rvec = iota16 + b0
                for d in range(EMBED_DIM):
                    cvec = jnp.full((16,), d, jnp.int32)
                    v = plsc.load_gather(rows_v, [rvec, cvec])
                    w_v[d // 8, d % 8, pl.ds(b0, 16)] = v
            outs = []
            for d4 in range(EMBED_DIM // 8):
                outs.append(
                    pltpu.async_copy(
                        w_v.at[d4], out_hbm.at[f, d4, bb], osem))
            for cp in outs:
                cp.wait()
            return 0

        lax.fori_loop(0, UPW, unit_body, 0)

    return gather_kernel


def kernel(x, embeddings):
    batch, n_fields = x.shape
    idxT = x.T.astype(jnp.int32).reshape(N_UNITS, BBLK)
    table_rm = _tc_transpose(embeddings.T)
    out5 = _build_gather()(idxT, table_rm)
    out = out5.transpose(2, 4, 0, 1, 3).reshape(batch, n_fields, EMBED_DIM)
    return out


# speedup vs baseline: 1.1917x; 1.1917x over previous
import functools

import jax
import jax.numpy as jnp
from jax import lax
from jax.experimental import pallas as pl
from jax.experimental.pallas import tpu as pltpu
from jax.experimental.pallas import tpu_sc as plsc

EMBED_DIM = 32
VOCAB_ROWS = 1000000
BATCH = 16384
N_FIELDS = 26
CHUNK = 1024
NUM_WORKERS = 32
TBLK = 32768


def _eye():
    return (
        lax.broadcasted_iota(jnp.int32, (EMBED_DIM, EMBED_DIM), 0)
        == lax.broadcasted_iota(jnp.int32, (EMBED_DIM, EMBED_DIM), 1)
    ).astype(jnp.float32)


def _tc_transpose(table_t):
    v = table_t.shape[1]

    def body(in_ref, out_ref):
        out_ref[...] = lax.dot_general(
            in_ref[...], _eye(), (((0,), (0,)), ((), ())),
            preferred_element_type=jnp.float32,
        )

    return pl.pallas_call(
        body,
        grid=(pl.cdiv(v, TBLK),),
        in_specs=[pl.BlockSpec((EMBED_DIM, TBLK), lambda k: (0, k))],
        out_specs=pl.BlockSpec((TBLK, EMBED_DIM), lambda k: (k, 0)),
        out_shape=jax.ShapeDtypeStruct((v, EMBED_DIM), jnp.float32),
    )(table_t)


def _tc_to_output_layout(rows):
    def body(in_ref, out_ref):
        out_ref[0] = lax.dot_general(
            _eye(), in_ref[...], (((1,), (1,)), ((), ())),
            preferred_element_type=jnp.float32,
        )

    return pl.pallas_call(
        body,
        grid=(N_FIELDS,),
        in_specs=[pl.BlockSpec((BATCH, EMBED_DIM), lambda f: (f, 0))],
        out_specs=pl.BlockSpec((1, EMBED_DIM, BATCH), lambda f: (f, 0, 0)),
        out_shape=jax.ShapeDtypeStruct(
            (N_FIELDS, EMBED_DIM, BATCH), jnp.float32),
    )(rows)


def _build_gather(total_rows: int):
    n_chunks = total_rows // CHUNK
    cpw = n_chunks // NUM_WORKERS

    mesh = plsc.VectorSubcoreMesh(core_axis_name="c", subcore_axis_name="s")

    @functools.partial(
        pl.kernel,
        mesh=mesh,
        compiler_params=pltpu.CompilerParams(use_tc_tiling_on_sc=False),
        out_type=jax.ShapeDtypeStruct((total_rows, EMBED_DIM), jnp.float32),
        scratch_types=[
            pltpu.VMEM((cpw, CHUNK), jnp.int32),
            pltpu.VMEM((CHUNK, EMBED_DIM), jnp.float32),
            pltpu.SemaphoreType.DMA,
        ],
    )
    def gather_kernel(idx_hbm, table_hbm, out_hbm, idx_v, rows_v, gsem):
        wid = lax.axis_index("s") * 2 + lax.axis_index("c")
        cbase = wid * cpw
        pltpu.sync_copy(idx_hbm.at[pl.ds(cbase, cpw)], idx_v)

        def group_body(g, _):
            pltpu.async_copy(
                table_hbm.at[idx_v.at[g]], rows_v, gsem).wait()
            pltpu.sync_copy(
                rows_v, out_hbm.at[pl.ds((cbase + g) * CHUNK, CHUNK)])
            return 0

        lax.fori_loop(0, cpw, group_body, 0)

    return gather_kernel


def kernel(x, embeddings):
    batch, n_fields = x.shape
    total = batch * n_fields
    idxT = x.T.astype(jnp.int32).reshape(total // CHUNK, CHUNK)
    table_rm = _tc_transpose(embeddings.T)
    rows = _build_gather(total)(idxT, table_rm)
    out3 = _tc_to_output_layout(rows)
    return out3.transpose(2, 0, 1)


# speedup vs baseline: 1.2293x; 1.0316x over previous
import functools

import jax
import jax.numpy as jnp
from jax import lax
from jax.experimental import pallas as pl
from jax.experimental.pallas import tpu as pltpu
from jax.experimental.pallas import tpu_sc as plsc

EMBED_DIM = 32
CHUNK = 1024
NUM_WORKERS = 32


def _build_gather(total_rows: int):
    n_chunks = total_rows // CHUNK
    cpw = n_chunks // NUM_WORKERS

    mesh = plsc.VectorSubcoreMesh(core_axis_name="c", subcore_axis_name="s")

    @functools.partial(
        pl.kernel,
        mesh=mesh,
        compiler_params=pltpu.CompilerParams(use_tc_tiling_on_sc=False),
        out_type=jax.ShapeDtypeStruct((total_rows, EMBED_DIM), jnp.float32),
        scratch_types=[
            pltpu.VMEM((cpw, CHUNK), jnp.int32),
            pltpu.VMEM((CHUNK, EMBED_DIM), jnp.float32),
            pltpu.SemaphoreType.DMA,
        ],
    )
    def gather_kernel(idx_hbm, table_hbm, out_hbm, idx_v, rows_v, gsem):
        wid = lax.axis_index("s") * 2 + lax.axis_index("c")
        cbase = wid * cpw
        pltpu.sync_copy(idx_hbm.at[pl.ds(cbase, cpw)], idx_v)

        def group_body(g, _):
            pltpu.async_copy(
                table_hbm.at[idx_v.at[g]], rows_v, gsem).wait()
            pltpu.sync_copy(
                rows_v, out_hbm.at[pl.ds((cbase + g) * CHUNK, CHUNK)])
            return 0

        lax.fori_loop(0, cpw, group_body, 0)

    return gather_kernel


def kernel(x, embeddings):
    batch, n_fields = x.shape
    total = batch * n_fields
    idx2d = x.reshape(total).astype(jnp.int32).reshape(total // CHUNK, CHUNK)
    out = _build_gather(total)(idx2d, embeddings)
    return out.reshape(batch, n_fields, EMBED_DIM)
